# baseline (speedup 1.0000x reference)
import functools

import jax
import jax.numpy as jnp
from jax import lax
from jax.experimental import pallas as pl
from jax.experimental.pallas import tpu as pltpu
from jax.experimental.pallas import tpu_sc as plsc

_GATHER_DNUMS = lax.GatherDimensionNumbers(
    offset_dims=(), collapsed_slice_dims=(0,), start_index_map=(0,))


def _permute(v, idx):
    return lax.gather(v, idx[:, None], _GATHER_DNUMS, (1,),
                      mode=lax.GatherScatterMode.PROMISE_IN_BOUNDS)


BATCH = 4096
D = 1000
L = 16
NC, NS = 2, 16
NW = NC * NS
BPW = BATCH // NW
C = 16
NCHUNK = BPW // C
NV = D // L
TAIL = D - NV * L


def _body(a_hbm, w_hbm, u_hbm, i_hbm, j_hbm, oi_hbm, oj_hbm,
          u_v, i_v, j_v, a_v, wi_v, wj_v, oi_v, oj_v, sem0, sem1, sem2):
    wid = lax.axis_index("s") * NC + lax.axis_index("c")
    base = wid * BPW
    pltpu.sync_copy(u_hbm.at[pl.ds(base, BPW)], u_v)
    pltpu.sync_copy(i_hbm.at[pl.ds(base, BPW)], i_v)
    pltpu.sync_copy(j_hbm.at[pl.ds(base, BPW)], j_v)

    lane = lax.iota(jnp.int32, L)
    tail_mask = lane >= (L - TAIL)
    zero = jnp.zeros((L,), jnp.float32)

    def chunk_fn(c, _):
        off = c * C
        uvec = u_v[pl.ds(off, C)]
        ivec = i_v[pl.ds(off, C)]
        jvec = j_v[pl.ds(off, C)]
        cp0 = pltpu.async_copy(a_hbm.at[uvec], a_v, sem0)
        cp1 = pltpu.async_copy(w_hbm.at[ivec], wi_v, sem1)
        cp2 = pltpu.async_copy(w_hbm.at[jvec], wj_v, sem2)
        cp0.wait()
        cp1.wait()
        cp2.wait()

        def row_fn(r, res):
            res_i, res_j = res

            def k_fn(k, acc):
                ai, aj = acc
                av = a_v[r, pl.ds(k * L, L)]
                wiv = wi_v[r, pl.ds(k * L, L)]
                wjv = wj_v[r, pl.ds(k * L, L)]
                return ai + av * wiv, aj + av * wjv

            ai, aj = lax.fori_loop(0, NV, k_fn, (zero, zero))
            av = a_v[r, pl.ds(D - L, L)]
            wiv = wi_v[r, pl.ds(D - L, L)]
            wjv = wj_v[r, pl.ds(D - L, L)]
            ai = ai + jnp.where(tail_mask, av * wiv, 0.0)
            aj = aj + jnp.where(tail_mask, av * wjv, 0.0)
            for s in (8, 4, 2, 1):
                perm = lane ^ s
                ai = ai + _permute(ai, perm)
                aj = aj + _permute(aj, perm)
            sel = lane == r
            return jnp.where(sel, ai, res_i), jnp.where(sel, aj, res_j)

        res_i, res_j = lax.fori_loop(0, C, row_fn, (zero, zero))
        oi_v[pl.ds(off, C)] = res_i
        oj_v[pl.ds(off, C)] = res_j
        return 0

    lax.fori_loop(0, NCHUNK, chunk_fn, 0)
    pltpu.sync_copy(oi_v, oi_hbm.at[pl.ds(base, BPW)])
    pltpu.sync_copy(oj_v, oj_hbm.at[pl.ds(base, BPW)])


def kernel(A, W, user, item_i, item_j):
    user = user.astype(jnp.int32)
    item_i = item_i.astype(jnp.int32)
    item_j = item_j.astype(jnp.int32)
    mesh = plsc.VectorSubcoreMesh(core_axis_name="c", subcore_axis_name="s")
    f32 = jnp.float32
    run = pl.kernel(
        _body,
        out_type=(jax.ShapeDtypeStruct((BATCH,), f32),
                  jax.ShapeDtypeStruct((BATCH,), f32)),
        mesh=mesh,
        compiler_params=pltpu.CompilerParams(use_tc_tiling_on_sc=False),
        scratch_types=[
            pltpu.VMEM((BPW,), jnp.int32),
            pltpu.VMEM((BPW,), jnp.int32),
            pltpu.VMEM((BPW,), jnp.int32),
            pltpu.VMEM((C, D), f32),
            pltpu.VMEM((C, D), f32),
            pltpu.VMEM((C, D), f32),
            pltpu.VMEM((BPW,), f32),
            pltpu.VMEM((BPW,), f32),
            pltpu.SemaphoreType.DMA,
            pltpu.SemaphoreType.DMA,
            pltpu.SemaphoreType.DMA,
        ],
    )
    return run(A, W, user, item_i, item_j)


# speedup vs baseline: 3.6349x; 3.6349x over previous
import functools

import jax
import jax.numpy as jnp
from jax import lax
from jax.experimental import pallas as pl
from jax.experimental.pallas import tpu as pltpu
from jax.experimental.pallas import tpu_sc as plsc

BATCH = 4096
D = 1000
DP = 1024
L = 16
NC, NS = 2, 16
NW = NC * NS
BPW = BATCH // NW
C = 16
NCHUNK = BPW // C
NV = DP // L

R = 16
GRID = BATCH // R

_GATHER_DNUMS = lax.GatherDimensionNumbers(
    offset_dims=(), collapsed_slice_dims=(0,), start_index_map=(0,))


def _permute(v, idx):
    return lax.gather(v, idx[:, None], _GATHER_DNUMS, (1,),
                      mode=lax.GatherScatterMode.PROMISE_IN_BOUNDS)


def _tc_gather_body(u_ref, *refs):
    a_refs = refs[:R]
    o_ref = refs[R]
    b = pl.program_id(0)
    o_ref[:, pl.ds(D, DP - D)] = jnp.zeros((R, DP - D), jnp.float32)
    for k in range(R):
        rm = u_ref[b * R + k] % 8
        o_ref[k, pl.ds(0, D)] = a_refs[k][rm, :]


def _gather_a_rows(A, user):
    in_specs = [pl.BlockSpec((8, D), functools.partial(
        lambda k, b, u_ref: (u_ref[b * R + k] // 8, 0), k)) for k in range(R)]
    grid_spec = pltpu.PrefetchScalarGridSpec(
        num_scalar_prefetch=1,
        grid=(GRID,),
        in_specs=in_specs,
        out_specs=pl.BlockSpec((R, DP), lambda b, u_ref: (b, 0)),
    )
    return pl.pallas_call(
        _tc_gather_body,
        grid_spec=grid_spec,
        out_shape=jax.ShapeDtypeStruct((BATCH, DP), jnp.float32),
    )(user, *([A] * R))


def _sc_body(ar_hbm, w_hbm, i_hbm, j_hbm, oi_hbm, oj_hbm,
             i_v, j_v, a_v, wi_v, wj_v, oi_v, oj_v, sem0, sem1, sem2):
    wid = lax.axis_index("s") * NC + lax.axis_index("c")
    base = wid * BPW
    pltpu.sync_copy(i_hbm.at[pl.ds(base, BPW)], i_v)
    pltpu.sync_copy(j_hbm.at[pl.ds(base, BPW)], j_v)

    lane = lax.iota(jnp.int32, L)
    zero = jnp.zeros((L,), jnp.float32)

    def chunk_fn(c, _):
        off = c * C
        ivec = i_v[pl.ds(off, C)]
        jvec = j_v[pl.ds(off, C)]
        cp0 = pltpu.async_copy(ar_hbm.at[pl.ds(base + off, C)], a_v, sem0)
        cp1 = pltpu.async_copy(w_hbm.at[ivec], wi_v, sem1)
        cp2 = pltpu.async_copy(w_hbm.at[jvec], wj_v, sem2)
        cp0.wait()
        cp1.wait()
        cp2.wait()

        def row_fn(r, res):
            res_i, res_j = res

            def k_fn(k, acc):
                ai, aj = acc
                for t in range(4):
                    col = (k * 4 + t) * L
                    av = a_v[r, pl.ds(col, L)]
                    ai = ai + av * wi_v[r, pl.ds(col, L)]
                    aj = aj + av * wj_v[r, pl.ds(col, L)]
                return ai, aj

            ai, aj = lax.fori_loop(0, NV // 4, k_fn, (zero, zero))
            for s in (8, 4, 2, 1):
                perm = lane ^ s
                ai = ai + _permute(ai, perm)
                aj = aj + _permute(aj, perm)
            sel = lane == r
            return jnp.where(sel, ai, res_i), jnp.where(sel, aj, res_j)

        res_i, res_j = lax.fori_loop(0, C, row_fn, (zero, zero))
        oi_v[pl.ds(off, C)] = res_i
        oj_v[pl.ds(off, C)] = res_j
        return 0

    lax.fori_loop(0, NCHUNK, chunk_fn, 0)
    pltpu.sync_copy(oi_v, oi_hbm.at[pl.ds(base, BPW)])
    pltpu.sync_copy(oj_v, oj_hbm.at[pl.ds(base, BPW)])


def kernel(A, W, user, item_i, item_j):
    user = user.astype(jnp.int32)
    item_i = item_i.astype(jnp.int32)
    item_j = item_j.astype(jnp.int32)
    ar = _gather_a_rows(A, user)
    w_pad = jnp.pad(W, ((0, 0), (0, DP - D)))
    mesh = plsc.VectorSubcoreMesh(core_axis_name="c", subcore_axis_name="s")
    f32 = jnp.float32
    run = pl.kernel(
        _sc_body,
        out_type=(jax.ShapeDtypeStruct((BATCH,), f32),
                  jax.ShapeDtypeStruct((BATCH,), f32)),
        mesh=mesh,
        scratch_types=[
            pltpu.VMEM((BPW,), jnp.int32),
            pltpu.VMEM((BPW,), jnp.int32),
            pltpu.VMEM((C, DP), f32),
            pltpu.VMEM((C, DP), f32),
            pltpu.VMEM((C, DP), f32),
            pltpu.VMEM((BPW,), f32),
            pltpu.VMEM((BPW,), f32),
            pltpu.SemaphoreType.DMA,
            pltpu.SemaphoreType.DMA,
            pltpu.SemaphoreType.DMA,
        ],
    )
    return run(ar, w_pad, item_i, item_j)


# speedup vs baseline: 3.7363x; 1.0279x over previous
import functools

import jax
import jax.numpy as jnp
from jax import lax
from jax.experimental import pallas as pl
from jax.experimental.pallas import tpu as pltpu
from jax.experimental.pallas import tpu_sc as plsc

BATCH = 4096
D = 1000
DP = 1024
L = 16
NC, NS = 2, 16
NW = NC * NS
BPW = BATCH // NW
C = 16
NCHUNK = BPW // C
NV = DP // L

R = 32
GRID = BATCH // R

_GATHER_DNUMS = lax.GatherDimensionNumbers(
    offset_dims=(), collapsed_slice_dims=(0,), start_index_map=(0,))


def _permute(v, idx):
    return lax.gather(v, idx[:, None], _GATHER_DNUMS, (1,),
                      mode=lax.GatherScatterMode.PROMISE_IN_BOUNDS)


def _tc_gather_body(u_ref, *refs):
    a_refs = refs[:R]
    o_ref = refs[R]
    b = pl.program_id(0)
    o_ref[:, pl.ds(D, DP - D)] = jnp.zeros((R, DP - D), jnp.float32)
    row_iota = lax.broadcasted_iota(jnp.int32, (8, D), 0)
    for k in range(R):
        rm = u_ref[b * R + k] % 8
        rows8 = a_refs[k][:, :]
        picked = jnp.where(row_iota == rm, rows8, 0.0)
        o_ref[k, pl.ds(0, D)] = jnp.sum(picked, axis=0)


def _gather_a_rows(A, user):
    in_specs = [pl.BlockSpec((8, D), functools.partial(
        lambda k, b, u_ref: (u_ref[b * R + k] // 8, 0), k)) for k in range(R)]
    grid_spec = pltpu.PrefetchScalarGridSpec(
        num_scalar_prefetch=1,
        grid=(GRID,),
        in_specs=in_specs,
        out_specs=pl.BlockSpec((R, DP), lambda b, u_ref: (b, 0)),
    )
    return pl.pallas_call(
        _tc_gather_body,
        grid_spec=grid_spec,
        out_shape=jax.ShapeDtypeStruct((BATCH, DP), jnp.float32),
    )(user, *([A] * R))


def _sc_body(ar_hbm, w_hbm, i_hbm, j_hbm, oi_hbm, oj_hbm,
             i_v, j_v, a_v, wi_v, wj_v, oi_v, oj_v, sem0, sem1, sem2):
    wid = lax.axis_index("s") * NC + lax.axis_index("c")
    base = wid * BPW
    pltpu.sync_copy(i_hbm.at[pl.ds(base, BPW)], i_v)
    pltpu.sync_copy(j_hbm.at[pl.ds(base, BPW)], j_v)

    lane = lax.iota(jnp.int32, L)
    zero = jnp.zeros((L,), jnp.float32)

    def chunk_fn(c, _):
        off = c * C
        ivec = i_v[pl.ds(off, C)]
        jvec = j_v[pl.ds(off, C)]
        cp0 = pltpu.async_copy(ar_hbm.at[pl.ds(base + off, C)], a_v, sem0)
        cp1 = pltpu.async_copy(w_hbm.at[ivec], wi_v, sem1)
        cp2 = pltpu.async_copy(w_hbm.at[jvec], wj_v, sem2)
        cp0.wait()
        cp1.wait()
        cp2.wait()

        def row_fn(r, res):
            res_i, res_j = res

            def k_fn(k, acc):
                ai, aj = acc
                for t in range(4):
                    col = (k * 4 + t) * L
                    av = a_v[r, pl.ds(col, L)]
                    ai = ai + av * wi_v[r, pl.ds(col, L)]
                    aj = aj + av * wj_v[r, pl.ds(col, L)]
                return ai, aj

            ai, aj = lax.fori_loop(0, NV // 4, k_fn, (zero, zero))
            for s in (8, 4, 2, 1):
                perm = lane ^ s
                ai = ai + _permute(ai, perm)
                aj = aj + _permute(aj, perm)
            sel = lane == r
            return jnp.where(sel, ai, res_i), jnp.where(sel, aj, res_j)

        res_i, res_j = lax.fori_loop(0, C, row_fn, (zero, zero))
        oi_v[pl.ds(off, C)] = res_i
        oj_v[pl.ds(off, C)] = res_j
        return 0

    lax.fori_loop(0, NCHUNK, chunk_fn, 0)
    pltpu.sync_copy(oi_v, oi_hbm.at[pl.ds(base, BPW)])
    pltpu.sync_copy(oj_v, oj_hbm.at[pl.ds(base, BPW)])


def kernel(A, W, user, item_i, item_j):
    user = user.astype(jnp.int32)
    item_i = item_i.astype(jnp.int32)
    item_j = item_j.astype(jnp.int32)
    ar = _gather_a_rows(A, user)
    w_pad = jnp.pad(W, ((0, 0), (0, DP - D)))
    mesh = plsc.VectorSubcoreMesh(core_axis_name="c", subcore_axis_name="s")
    f32 = jnp.float32
    run = pl.kernel(
        _sc_body,
        out_type=(jax.ShapeDtypeStruct((BATCH,), f32),
                  jax.ShapeDtypeStruct((BATCH,), f32)),
        mesh=mesh,
        scratch_types=[
            pltpu.VMEM((BPW,), jnp.int32),
            pltpu.VMEM((BPW,), jnp.int32),
            pltpu.VMEM((C, DP), f32),
            pltpu.VMEM((C, DP), f32),
            pltpu.VMEM((C, DP), f32),
            pltpu.VMEM((BPW,), f32),
            pltpu.VMEM((BPW,), f32),
            pltpu.SemaphoreType.DMA,
            pltpu.SemaphoreType.DMA,
            pltpu.SemaphoreType.DMA,
        ],
    )
    return run(ar, w_pad, item_i, item_j)


# speedup vs baseline: 4.7896x; 1.2819x over previous
import functools

import jax
import jax.numpy as jnp
from jax import lax
from jax.experimental import pallas as pl
from jax.experimental.pallas import tpu as pltpu
from jax.experimental.pallas import tpu_sc as plsc

BATCH = 4096
D = 1000
DP = 1024
L = 16
NC, NS = 2, 16
NW = NC * NS
BPW = BATCH // NW
C = 8
NCHUNK = BPW // C
NV = D // L
TAIL = D - NV * L

_GATHER_DNUMS = lax.GatherDimensionNumbers(
    offset_dims=(), collapsed_slice_dims=(0,), start_index_map=(0,))


def _permute(v, idx):
    return lax.gather(v, idx[:, None], _GATHER_DNUMS, (1,),
                      mode=lax.GatherScatterMode.PROMISE_IN_BOUNDS)


def _sc_body(a_hbm, w_hbm, u_hbm, i_hbm, j_hbm, oi_hbm, oj_hbm,
             u_v, i_v, j_v, a_v, wi_v, wj_v, oi_v, oj_v,
             sem_a, sem_w):
    wid = lax.axis_index("s") * NC + lax.axis_index("c")
    base = wid * BPW
    pltpu.sync_copy(u_hbm.at[pl.ds(base, BPW)], u_v.at[pl.ds(0, BPW)])
    pltpu.sync_copy(i_hbm.at[pl.ds(base, BPW)], i_v)
    pltpu.sync_copy(j_hbm.at[pl.ds(base, BPW)], j_v)

    lane = lax.iota(jnp.int32, L)
    tail_mask = lane >= (L - TAIL)
    zero = jnp.zeros((L,), jnp.float32)

    def chunk_fn(c, chunk_res):
        off = c * C
        parity = lax.rem(c, 2)
        cpw1 = pltpu.async_copy(w_hbm.at[i_v.at[pl.ds(off, C)]], wi_v, sem_w)
        cpw2 = pltpu.async_copy(w_hbm.at[j_v.at[pl.ds(off, C)]], wj_v, sem_w)
        uvec = u_v[pl.ds(off, L)]
        us = [uvec[r] for r in range(C)]
        rms = [lax.rem(u, 8) for u in us]
        row_cps = []
        for r in range(C):
            g8 = pl.multiple_of(us[r] - rms[r], 8)
            cp = pltpu.async_copy(a_hbm.at[pl.ds(g8, 8)], a_v.at[r], sem_a)
            row_cps.append(cp)
        for cp in row_cps:
            cp.wait()
        cpw1.wait()
        cpw2.wait()

        res_i, res_j = chunk_res
        for r in range(C):
            rm = rms[r]

            def k_fn(k, acc, r=r, rm=rm):
                ai, aj = acc
                for t in range(2):
                    col = (k * 2 + t) * L
                    av = a_v[r, rm, pl.ds(col, L)]
                    ai = ai + av * wi_v[r, pl.ds(col, L)]
                    aj = aj + av * wj_v[r, pl.ds(col, L)]
                return ai, aj

            ai, aj = lax.fori_loop(0, NV // 2, k_fn, (zero, zero))
            av = a_v[r, rm, pl.ds(D - L, L)]
            ai = ai + jnp.where(tail_mask, av * wi_v[r, pl.ds(D - L, L)], 0.0)
            aj = aj + jnp.where(tail_mask, av * wj_v[r, pl.ds(D - L, L)], 0.0)
            for s in (8, 4, 2, 1):
                perm = lane ^ s
                ai = ai + _permute(ai, perm)
                aj = aj + _permute(aj, perm)
            sel = lane == (r + parity * C)
            res_i = jnp.where(sel, ai, res_i)
            res_j = jnp.where(sel, aj, res_j)

        @pl.when(parity == 1)
        def _store():
            st = (c - 1) * C
            oi_v[pl.ds(st, 2 * C)] = res_i
            oj_v[pl.ds(st, 2 * C)] = res_j

        keep = parity == 0
        return (jnp.where(keep, res_i, zero), jnp.where(keep, res_j, zero))

    lax.fori_loop(0, NCHUNK, chunk_fn, (zero, zero))
    pltpu.sync_copy(oi_v, oi_hbm.at[pl.ds(base, BPW)])
    pltpu.sync_copy(oj_v, oj_hbm.at[pl.ds(base, BPW)])


def kernel(A, W, user, item_i, item_j):
    user = user.astype(jnp.int32)
    item_i = item_i.astype(jnp.int32)
    item_j = item_j.astype(jnp.int32)
    w_pad = jnp.pad(W, ((0, 0), (0, DP - D)))
    mesh = plsc.VectorSubcoreMesh(core_axis_name="c", subcore_axis_name="s")
    f32 = jnp.float32
    run = pl.kernel(
        _sc_body,
        out_type=(jax.ShapeDtypeStruct((BATCH,), f32),
                  jax.ShapeDtypeStruct((BATCH,), f32)),
        mesh=mesh,
        scratch_types=[
            pltpu.VMEM((BPW + L - C,), jnp.int32),
            pltpu.VMEM((BPW,), jnp.int32),
            pltpu.VMEM((BPW,), jnp.int32),
            pltpu.VMEM((C, 8, D), f32),
            pltpu.VMEM((C, DP), f32),
            pltpu.VMEM((C, DP), f32),
            pltpu.VMEM((BPW,), f32),
            pltpu.VMEM((BPW,), f32),
            pltpu.SemaphoreType.DMA,
            pltpu.SemaphoreType.DMA,
        ],
    )
    return run(A, w_pad, user, item_i, item_j)
